# baseline (speedup 1.0000x reference)
import functools

import jax
import jax.numpy as jnp
from jax import lax
from jax.experimental import pallas as pl
from jax.experimental.pallas import tpu as pltpu

_INTERP = False

H = 128
EB = 512
NB = 2000


def _embed_body(x_ref, emb_ref, embb_ref, w1a_ref, b1_ref, w1b_ref,
                h_ref, p_ref, q_ref):
    x = x_ref[:, 0]
    t = x / jnp.max(x)
    e0 = emb_ref[0, :][None, :]
    e1 = emb_ref[1, :][None, :]
    e2 = emb_ref[2, :][None, :]
    h = embb_ref[0, :][None, :] + e0 + t[:, None] * e1 + (t * t)[:, None] * e2
    h_ref[:, :] = h
    p_ref[:, :] = jnp.dot(h, w1a_ref[:, :], preferred_element_type=jnp.float32) \
        + b1_ref[0, :][None, :]
    q_ref[:, :] = jnp.dot(h, w1b_ref[:, :], preferred_element_type=jnp.float32)


def _embed_call(x2, emb_W, emb_b, w1a, b1, w1b):
    n = x2.shape[0]
    return pl.pallas_call(
        _embed_body,
        out_shape=[jax.ShapeDtypeStruct((n, H), jnp.float32)] * 3,
        interpret=_INTERP,
    )(x2, emb_W, emb_b, w1a, b1, w1b)


def _edge_body(pd_ref, qs_ref, rd_ref, rs_ref, w1c_ref, w2_ref, b2_ref,
               wr1_ref, br1_ref, wr2_ref, br2_ref, mij_ref, delr_ref):
    dr = rs_ref[:, :] - rd_ref[:, :]
    norm = jnp.sum(dr * dr, axis=1)
    x1 = pd_ref[:, :] + qs_ref[:, :] + norm[:, None] * w1c_ref[0, :][None, :]
    a = jax.nn.silu(x1)
    mij = jax.nn.silu(
        jnp.dot(a, w2_ref[:, :], preferred_element_type=jnp.float32)
        + b2_ref[0, :][None, :])
    rr = jax.nn.silu(
        jnp.dot(mij, wr1_ref[:, :], preferred_element_type=jnp.float32)
        + br1_ref[0, :][None, :])
    tt = jnp.tanh(jnp.sum(rr * wr2_ref[0, :][None, :], axis=1) + br2_ref[0, 0])
    mij_ref[:, :] = mij
    delr_ref[:, :] = dr * tt[:, None]


def _edge_call(pd, qs, rd, rs, w1c, w2, b2, wr1, br1, wr2, br2):
    ne = pd.shape[0]
    grid = ne // EB
    full = lambda a, b: pl.BlockSpec((a, b), lambda i: (0, 0))
    return pl.pallas_call(
        _edge_body,
        grid=(grid,),
        in_specs=[
            pl.BlockSpec((EB, H), lambda i: (i, 0)),
            pl.BlockSpec((EB, H), lambda i: (i, 0)),
            pl.BlockSpec((EB, 4), lambda i: (i, 0)),
            pl.BlockSpec((EB, 4), lambda i: (i, 0)),
            full(1, H), full(H, H), full(1, H),
            full(H, H), full(1, H), full(1, H), full(1, 1),
        ],
        out_specs=[
            pl.BlockSpec((EB, H), lambda i: (i, 0)),
            pl.BlockSpec((EB, 4), lambda i: (i, 0)),
        ],
        out_shape=[
            jax.ShapeDtypeStruct((ne, H), jnp.float32),
            jax.ShapeDtypeStruct((ne, 4), jnp.float32),
        ],
        interpret=_INTERP,
    )(pd, qs, rd, rs, w1c, w2, b2, wr1, br1, wr2, br2)


def _node_body(has_next, h_ref, r_ref, m_ref, d_ref,
               wh1a_ref, wh1b_ref, bh1_ref, wh2_ref, bh2_ref,
               w1a_ref, b1_ref, w1b_ref,
               h_out, r_out, p_out, q_out):
    h = h_ref[:, :]
    m = m_ref[:, :]
    a = jax.nn.silu(
        jnp.dot(h, wh1a_ref[:, :], preferred_element_type=jnp.float32)
        + jnp.dot(m, wh1b_ref[:, :], preferred_element_type=jnp.float32)
        + bh1_ref[0, :][None, :])
    hp = jnp.dot(a, wh2_ref[:, :], preferred_element_type=jnp.float32) \
        + bh2_ref[0, :][None, :]
    mask = jnp.sum(jnp.abs(m), axis=1) > 0.0
    h_new = jnp.where(mask[:, None], hp, h)
    mask2 = jnp.sum(jnp.abs(h_new), axis=1) == 0.0
    h_val = h_new + jnp.where(mask2[:, None], h, 0.0)
    h_out[:, :] = h_val
    r_out[:, :] = r_ref[:, :] + d_ref[:, :]
    if has_next:
        p_out[:, :] = jnp.dot(h_val, w1a_ref[:, :],
                              preferred_element_type=jnp.float32) \
            + b1_ref[0, :][None, :]
        q_out[:, :] = jnp.dot(h_val, w1b_ref[:, :],
                              preferred_element_type=jnp.float32)
    else:
        p_out[:, :] = jnp.zeros_like(h_val)
        q_out[:, :] = jnp.zeros_like(h_val)


def _node_call(h, r4, m, d, wh1a, wh1b, bh1, wh2, bh2, nxt):
    n = h.shape[0]
    grid = n // NB
    has_next = nxt is not None
    if not has_next:
        nxt = (jnp.zeros((H, H), jnp.float32), jnp.zeros((1, H), jnp.float32),
               jnp.zeros((H, H), jnp.float32))
    w1a, b1, w1b = nxt
    full = lambda a, b: pl.BlockSpec((a, b), lambda i: (0, 0))
    return pl.pallas_call(
        functools.partial(_node_body, has_next),
        grid=(grid,),
        in_specs=[
            pl.BlockSpec((NB, H), lambda i: (i, 0)),
            pl.BlockSpec((NB, 4), lambda i: (i, 0)),
            pl.BlockSpec((NB, H), lambda i: (i, 0)),
            pl.BlockSpec((NB, 4), lambda i: (i, 0)),
            full(H, H), full(H, H), full(1, H), full(H, H), full(1, H),
            full(H, H), full(1, H), full(H, H),
        ],
        out_specs=[
            pl.BlockSpec((NB, H), lambda i: (i, 0)),
            pl.BlockSpec((NB, 4), lambda i: (i, 0)),
            pl.BlockSpec((NB, H), lambda i: (i, 0)),
            pl.BlockSpec((NB, H), lambda i: (i, 0)),
        ],
        out_shape=[
            jax.ShapeDtypeStruct((n, H), jnp.float32),
            jax.ShapeDtypeStruct((n, 4), jnp.float32),
            jax.ShapeDtypeStruct((n, H), jnp.float32),
            jax.ShapeDtypeStruct((n, H), jnp.float32),
        ],
        interpret=_INTERP,
    )(h, r4, m, d, wh1a, wh1b, bh1, wh2, bh2, w1a, b1, w1b)


def _final_body(h_ref, c_ref, w1_ref, b1_ref, w2_ref, b2_ref, w3_ref, b3_ref,
                out_ref):
    h = h_ref[:, :]
    e = jax.nn.silu(
        jnp.dot(h, w1_ref[:, :], preferred_element_type=jnp.float32)
        + b1_ref[0, :][None, :])
    e = jax.nn.silu(
        jnp.dot(e, w2_ref[:, :], preferred_element_type=jnp.float32)
        + b2_ref[0, :][None, :])
    ev = jnp.sum(e * w3_ref[0, :][None, :], axis=1) + b3_ref[0, 0]
    cmask = (c_ref[:, 0] == 0).astype(jnp.float32)
    out_ref[:, :] = jnp.sum(ev * cmask)[None, None]


def _final_call(h, c2, w1, b1, w2, b2, w3, b3):
    return pl.pallas_call(
        _final_body,
        out_shape=jax.ShapeDtypeStruct((1, 1), jnp.float32),
        interpret=_INTERP,
    )(h, c2, w1, b1, w2, b2, w3, b3)


def kernel(x, r, edge_index0, edge_index1, edge_index2, contributions, params):
    n = x.shape[0]
    r4 = jnp.pad(r, ((0, 0), (0, 1)))
    x2 = x[:, None]
    layers = params['layers']

    def wsplit(p):
        w1 = p['e_W1']
        return (w1[:H], w1[H:2 * H], w1[2 * H:2 * H + 1], p['e_b1'][None, :])

    w1a0, w1b0, _, b10 = wsplit(layers[0])
    h, P, Q = _embed_call(x2, params['emb_W'], params['emb_b'][None, :],
                          w1a0, b10, w1b0)

    edges = [edge_index2, edge_index1, edge_index0]
    for li in range(3):
        p = layers[li]
        src = edges[li][0]
        dst = edges[li][1]
        _, _, w1c, _ = wsplit(p)
        pd = P[dst]
        qs = Q[src]
        rd = r4[dst]
        rs = r4[src]
        mij, delr = _edge_call(
            pd, qs, rd, rs, w1c,
            p['e_W2'], p['e_b2'][None, :],
            p['r_W1'], p['r_b1'][None, :],
            p['r_W2'].T, p['r_b2'][None, :])
        m = jax.ops.segment_sum(mij, dst, num_segments=n)
        d = jax.ops.segment_sum(delr, dst, num_segments=n)
        nxt = None
        if li < 2:
            w1a_n, w1b_n, _, b1_n = wsplit(layers[li + 1])
            nxt = (w1a_n, b1_n, w1b_n)
        wh1 = p['h_W1']
        h, r4, P, Q = _node_call(
            h, r4, m, d, wh1[:H], wh1[H:], p['h_b1'][None, :],
            p['h_W2'], p['h_b2'][None, :], nxt)

    return _final_call(h, contributions[:, None],
                       params['mlp_W1'], params['mlp_b1'][None, :],
                       params['mlp_W2'], params['mlp_b2'][None, :],
                       params['mlp_W3'].T, params['mlp_b3'][None, :])
